# baseline (speedup 1.0000x reference)
import functools

import jax
import jax.numpy as jnp
from jax import lax
from jax.experimental import pallas as pl
from jax.experimental.pallas import tpu as pltpu
from jax.experimental.pallas import tpu_sc as plsc

N = 10000
NINP = 128
E = 320000

NC, NS = 2, 16
CHUNK = 64
IBLK = 8
ACC_ROWS = 10240
TRASH = N
DEG_W = 16

ROW_BLK = 400

_mesh = lambda: plsc.VectorSubcoreMesh(core_axis_name="c", subcore_axis_name="s")


def _edge_pass(nchunks, width, chunk):
    assert nchunks % 8 == 2 and nchunks >= 10
    ngroups = (nchunks - 2) // 8
    cparams = (pltpu.CompilerParams(use_tc_tiling_on_sc=False)
               if width < 128 else None)

    @functools.partial(
        pl.kernel,
        mesh=_mesh(),
        compiler_params=cparams,
        out_type=jax.ShapeDtypeStruct((NC, ACC_ROWS, width), jnp.float32),
        scratch_types=[
            pltpu.VMEM((2, IBLK, chunk), jnp.int32),
            pltpu.VMEM((2, IBLK, chunk), jnp.int32),
            pltpu.VMEM((4, chunk, width), jnp.float32),
            pltpu.VMEM_SHARED((ACC_ROWS, width), jnp.float32),
            [pltpu.SemaphoreType.DMA] * 4,
            [pltpu.SemaphoreType.DMA] * 4,
            [pltpu.SemaphoreType.DMA] * 2,
        ],
    )
    def k(src_hbm, dst_hbm, table_hbm, out_hbm, src_blk, dst_blk, rows_v,
          acc, gsem, ssem, isem):
        c = lax.axis_index("c")
        s = lax.axis_index("s")
        rows_per_tile = ACC_ROWS // NS

        def zrow(i, carry):
            for jj in range(width // 16):
                rows_v[0, i, pl.ds(jj * 16, 16)] = jnp.zeros((16,),
                                                             jnp.float32)
            return carry
        lax.fori_loop(0, chunk, zrow, 0)
        for b in range(rows_per_tile // chunk):
            pltpu.sync_copy(rows_v.at[0],
                            acc.at[pl.ds(s * rows_per_tile + b * chunk,
                                         chunk)])
        plsc.subcore_barrier()

        def ib_start(blk, par):
            pltpu.async_copy(src_hbm.at[c, s, pl.ds(blk * IBLK, IBLK)],
                             src_blk.at[par], isem[0])
            pltpu.async_copy(dst_hbm.at[c, s, pl.ds(blk * IBLK, IBLK)],
                             dst_blk.at[par], isem[1])

        def ib_wait(par):
            pltpu.make_async_copy(src_hbm.at[c, s, pl.ds(0, IBLK)],
                                  src_blk.at[par], isem[0]).wait()
            pltpu.make_async_copy(dst_hbm.at[c, s, pl.ds(0, IBLK)],
                                  dst_blk.at[par], isem[1]).wait()

        def g_start(par, slot, b):
            pltpu.async_copy(table_hbm.at[src_blk.at[par, slot]],
                             rows_v.at[b], gsem[b])

        def g_wait(b):
            pltpu.make_async_copy(table_hbm.at[src_blk.at[0, 0]],
                                  rows_v.at[b], gsem[b]).wait()

        def s_start(par, slot, b):
            pltpu.async_copy(rows_v.at[b], acc.at[dst_blk.at[par, slot]],
                             ssem[b], add=True)

        def s_wait(b):
            pltpu.make_async_copy(rows_v.at[b], acc.at[dst_blk.at[0, 0]],
                                  ssem[b]).wait()

        zero = jnp.int32(0)
        ib_start(zero, zero)
        ib_wait(zero)
        g_start(zero, 0, 0)
        g_start(zero, 1, 1)
        g_start(zero, 2, 2); g_wait(0); s_start(zero, 0, 0)
        g_start(zero, 3, 3); g_wait(1); s_start(zero, 1, 1)

        def body(j, carry):
            parj = j & 1
            parj1 = (j + 1) & 1
            for p in range(8):
                bb = (2 + p) % 4
                if p == 0:
                    ib_start(j + 1, parj1)
                s_wait((bb + 2) % 4)
                if p == 4:
                    ib_wait(parj1)
                g_par = parj if p <= 3 else parj1
                g_start(g_par, (4 + p) % 8, (bb + 2) % 4)
                g_wait(bb)
                s_par = parj if p <= 5 else parj1
                s_start(s_par, (2 + p) % 8, bb)
            return carry
        lax.fori_loop(0, ngroups, body, 0)

        g_wait(2); g_wait(3)
        s_wait(0); s_wait(1)

        plsc.subcore_barrier()
        pltpu.sync_copy(acc.at[pl.ds(s * rows_per_tile, rows_per_tile)],
                        out_hbm.at[c, pl.ds(s * rows_per_tile, rows_per_tile)])

    return k


def _deg_pass(nchunks):
    assert nchunks % 2 == 0

    @functools.partial(
        pl.kernel,
        mesh=_mesh(),
        compiler_params=pltpu.CompilerParams(use_tc_tiling_on_sc=False),
        out_type=jax.ShapeDtypeStruct((NC, ACC_ROWS, DEG_W), jnp.float32),
        scratch_types=[
            pltpu.VMEM((nchunks, CHUNK), jnp.int32),
            pltpu.VMEM((CHUNK, DEG_W), jnp.float32),
            pltpu.VMEM_SHARED((ACC_ROWS, DEG_W), jnp.float32),
            [pltpu.SemaphoreType.DMA] * 2,
        ],
    )
    def k(dst_hbm, out_hbm, dst_all, ones_v, acc, ssem):
        c = lax.axis_index("c")
        s = lax.axis_index("s")
        rows_per_tile = ACC_ROWS // NS

        def fill(val):
            def frow(i, carry):
                for jj in range(DEG_W // 16):
                    ones_v[i, pl.ds(jj * 16, 16)] = jnp.full((16,), val,
                                                             jnp.float32)
                return carry
            lax.fori_loop(0, CHUNK, frow, 0)

        fill(0.0)
        for b in range(rows_per_tile // CHUNK):
            pltpu.sync_copy(
                ones_v, acc.at[pl.ds(s * rows_per_tile + b * CHUNK, CHUNK)])
        fill(1.0)
        pltpu.sync_copy(dst_hbm.at[c, s], dst_all)
        plsc.subcore_barrier()

        def body(j, carry):
            for p in range(2):
                pltpu.async_copy(ones_v, acc.at[dst_all.at[2 * j + p]],
                                 ssem[p], add=True)
            for p in range(2):
                pltpu.make_async_copy(ones_v, acc.at[dst_all.at[0]],
                                      ssem[p]).wait()
            return carry
        lax.fori_loop(0, nchunks // 2, body, 0)

        plsc.subcore_barrier()
        pltpu.sync_copy(acc.at[pl.ds(s * rows_per_tile, rows_per_tile)],
                        out_hbm.at[c, pl.ds(s * rows_per_tile, rows_per_tile)])

    return k


def _dis_of(degs_blk):
    deg = degs_blk[0, 0] + degs_blk[0, 1] + 1.0
    return lax.rsqrt(deg)[:, None]


_DEG_SPEC = lambda: pl.BlockSpec((1, NC, ROW_BLK), lambda i: (i, 0, 0))


def _tc_stage1(degs, emb, W1):
    def body(degs_ref, emb_ref, w_ref, g_ref):
        dis = _dis_of(degs_ref)
        h = jnp.dot(emb_ref[...], w_ref[...],
                    preferred_element_type=jnp.float32)
        g_ref[0] = h[:, :NINP] * dis
        g_ref[1] = h[:, NINP:] * dis

    return pl.pallas_call(
        body,
        grid=(N // ROW_BLK,),
        in_specs=[
            _DEG_SPEC(),
            pl.BlockSpec((ROW_BLK, NINP), lambda i: (i, 0)),
            pl.BlockSpec((NINP, 2 * NINP), lambda i: (0, 0)),
        ],
        out_specs=pl.BlockSpec((NC, ROW_BLK, NINP), lambda i: (0, i, 0)),
        out_shape=jax.ShapeDtypeStruct((NC, N, NINP), jnp.float32),
    )(degs, emb, W1)


def _tc_stage2(degs, s1, g1, W2, b1):
    def body(degs_ref, s1_ref, g1_ref, w_ref, b_ref, g2_ref):
        dis = _dis_of(degs_ref)
        t0 = (s1_ref[0] + g1_ref[0]) * dis
        t1 = (s1_ref[1] + g1_ref[1]) * dis
        x1 = jnp.concatenate([t0, t1], axis=1) + b_ref[...]
        g2 = jnp.dot(x1, w_ref[...],
                     preferred_element_type=jnp.float32) * dis
        g2_ref[0] = g2
        g2_ref[1] = g2

    return pl.pallas_call(
        body,
        grid=(N // ROW_BLK,),
        in_specs=[
            _DEG_SPEC(),
            pl.BlockSpec((NC, ROW_BLK, NINP), lambda i: (0, i, 0)),
            pl.BlockSpec((NC, ROW_BLK, NINP), lambda i: (0, i, 0)),
            pl.BlockSpec((2 * NINP, NINP), lambda i: (0, 0)),
            pl.BlockSpec((1, 2 * NINP), lambda i: (0, 0)),
        ],
        out_specs=pl.BlockSpec((NC, ROW_BLK, NINP), lambda i: (0, i, 0)),
        out_shape=jax.ShapeDtypeStruct((NC, N, NINP), jnp.float32),
    )(degs, s1, g1, W2, b1)


def _tc_stage3(degs, s2, g2, b2):
    def body(degs_ref, s2_ref, g2_ref, b_ref, out_ref):
        dis = _dis_of(degs_ref)
        out_ref[...] = (s2_ref[0] + s2_ref[1] + g2_ref[0]) * dis + b_ref[...]

    return pl.pallas_call(
        body,
        grid=(N // ROW_BLK,),
        in_specs=[
            _DEG_SPEC(),
            pl.BlockSpec((NC, ROW_BLK, NINP), lambda i: (0, i, 0)),
            pl.BlockSpec((NC, ROW_BLK, NINP), lambda i: (0, i, 0)),
            pl.BlockSpec((1, NINP), lambda i: (0, 0)),
        ],
        out_specs=pl.BlockSpec((ROW_BLK, NINP), lambda i: (i, 0)),
        out_shape=jax.ShapeDtypeStruct((N, NINP), jnp.float32),
    )(degs, s2, g2, b2)


def _ceil_div(a, b):
    return (a + b - 1) // b


def _round_chunks(n):
    while n % 8 != 2:
        n += 1
    return n


@jax.jit
def kernel(edge_index, emb, W1, b1, W2, b2):
    src = edge_index[0].astype(jnp.int32)
    dst = edge_index[1].astype(jnp.int32)

    def _trash(n):
        return TRASH + (jnp.arange(n, dtype=jnp.int32) % (ACC_ROWS - N))

    e_sub = E // NS
    c1 = _round_chunks(_ceil_div(e_sub, CHUNK))
    h1 = c1 + 6
    p1 = h1 * CHUNK - e_sub
    src1 = jnp.pad(src.reshape(NS, e_sub), ((0, 0), (0, p1)))
    src1 = src1.reshape(NS, h1, CHUNK)
    dst1 = jnp.concatenate(
        [dst.reshape(NS, e_sub),
         jnp.broadcast_to(_trash(p1), (NS, p1))], axis=1)
    dst1 = dst1.reshape(NS, h1, CHUNK)
    src_l1 = jnp.stack([src1, src1 + N])
    dst_l1 = jnp.stack([dst1, dst1])

    e_tile = E // (NC * NS)
    c2 = _round_chunks(_ceil_div(e_tile, CHUNK))
    h2 = c2 + 6
    p2 = h2 * CHUNK - e_tile
    src2 = jnp.pad(src.reshape(NC, NS, e_tile),
                   ((0, 0), (0, 0), (0, p2)))
    src2 = src2.reshape(NC, NS, h2, CHUNK)
    dst2 = jnp.concatenate(
        [dst.reshape(NC, NS, e_tile),
         jnp.broadcast_to(_trash(p2), (NC, NS, p2))], axis=2)
    dst2 = dst2.reshape(NC, NS, h2, CHUNK)

    degs = _deg_pass(h2)(dst2)[:, :N, 0]
    degs = degs.reshape(NC, N // ROW_BLK, ROW_BLK).transpose(1, 0, 2)
    g1 = _tc_stage1(degs, emb, W1)
    s1 = _edge_pass(c1, NINP, CHUNK)(src_l1, dst_l1,
                                     g1.reshape(NC * N, NINP))
    g2 = _tc_stage2(degs, s1, g1, W2, b1.reshape(1, -1))
    src2d = src2 + N * jnp.arange(NC, dtype=jnp.int32)[:, None, None, None]
    s2 = _edge_pass(c2, NINP, CHUNK)(src2d, dst2,
                                     g2.reshape(NC * N, NINP))
    return _tc_stage3(degs, s2, g2, b2.reshape(1, -1))


# speedup vs baseline: 1.0065x; 1.0065x over previous
import functools

import jax
import jax.numpy as jnp
from jax import lax
from jax.experimental import pallas as pl
from jax.experimental.pallas import tpu as pltpu
from jax.experimental.pallas import tpu_sc as plsc

N = 10000
NINP = 128
E = 320000

NC, NS = 2, 16
CHUNK = 80
IBLK = 8
ACC_ROWS = 10240
TRASH = N
DEG_W = 16

ROW_BLK = 400

_mesh = lambda: plsc.VectorSubcoreMesh(core_axis_name="c", subcore_axis_name="s")


def _edge_pass(nchunks, width, chunk):
    assert nchunks % 8 == 2 and nchunks >= 10
    ngroups = (nchunks - 2) // 8
    cparams = (pltpu.CompilerParams(use_tc_tiling_on_sc=False)
               if width < 128 else None)

    @functools.partial(
        pl.kernel,
        mesh=_mesh(),
        compiler_params=cparams,
        out_type=jax.ShapeDtypeStruct((NC, ACC_ROWS, width), jnp.float32),
        scratch_types=[
            pltpu.VMEM((2, IBLK, chunk), jnp.int32),
            pltpu.VMEM((2, IBLK, chunk), jnp.int32),
            pltpu.VMEM((4, chunk, width), jnp.float32),
            pltpu.VMEM_SHARED((ACC_ROWS, width), jnp.float32),
            [pltpu.SemaphoreType.DMA] * 4,
            [pltpu.SemaphoreType.DMA] * 4,
            [pltpu.SemaphoreType.DMA] * 2,
        ],
    )
    def k(src_hbm, dst_hbm, table_hbm, out_hbm, src_blk, dst_blk, rows_v,
          acc, gsem, ssem, isem):
        c = lax.axis_index("c")
        s = lax.axis_index("s")
        rows_per_tile = ACC_ROWS // NS

        def zrow(i, carry):
            for jj in range(width // 16):
                rows_v[0, i, pl.ds(jj * 16, 16)] = jnp.zeros((16,),
                                                             jnp.float32)
            return carry
        lax.fori_loop(0, chunk, zrow, 0)
        for b in range(rows_per_tile // chunk):
            pltpu.sync_copy(rows_v.at[0],
                            acc.at[pl.ds(s * rows_per_tile + b * chunk,
                                         chunk)])
        plsc.subcore_barrier()

        def ib_start(blk, par):
            pltpu.async_copy(src_hbm.at[c, s, pl.ds(blk * IBLK, IBLK)],
                             src_blk.at[par], isem[0])
            pltpu.async_copy(dst_hbm.at[c, s, pl.ds(blk * IBLK, IBLK)],
                             dst_blk.at[par], isem[1])

        def ib_wait(par):
            pltpu.make_async_copy(src_hbm.at[c, s, pl.ds(0, IBLK)],
                                  src_blk.at[par], isem[0]).wait()
            pltpu.make_async_copy(dst_hbm.at[c, s, pl.ds(0, IBLK)],
                                  dst_blk.at[par], isem[1]).wait()

        def g_start(par, slot, b):
            pltpu.async_copy(table_hbm.at[src_blk.at[par, slot]],
                             rows_v.at[b], gsem[b])

        def g_wait(b):
            pltpu.make_async_copy(table_hbm.at[src_blk.at[0, 0]],
                                  rows_v.at[b], gsem[b]).wait()

        def s_start(par, slot, b):
            pltpu.async_copy(rows_v.at[b], acc.at[dst_blk.at[par, slot]],
                             ssem[b], add=True)

        def s_wait(b):
            pltpu.make_async_copy(rows_v.at[b], acc.at[dst_blk.at[0, 0]],
                                  ssem[b]).wait()

        zero = jnp.int32(0)
        ib_start(zero, zero)
        ib_wait(zero)
        g_start(zero, 0, 0)
        g_start(zero, 1, 1)
        g_start(zero, 2, 2); g_wait(0); s_start(zero, 0, 0)
        g_start(zero, 3, 3); g_wait(1); s_start(zero, 1, 1)

        def body(j, carry):
            parj = j & 1
            parj1 = (j + 1) & 1
            for p in range(8):
                bb = (2 + p) % 4
                if p == 0:
                    ib_start(j + 1, parj1)
                s_wait((bb + 2) % 4)
                if p == 4:
                    ib_wait(parj1)
                g_par = parj if p <= 3 else parj1
                g_start(g_par, (4 + p) % 8, (bb + 2) % 4)
                g_wait(bb)
                s_par = parj if p <= 5 else parj1
                s_start(s_par, (2 + p) % 8, bb)
            return carry
        lax.fori_loop(0, ngroups, body, 0)

        g_wait(2); g_wait(3)
        s_wait(0); s_wait(1)

        plsc.subcore_barrier()
        pltpu.sync_copy(acc.at[pl.ds(s * rows_per_tile, rows_per_tile)],
                        out_hbm.at[c, pl.ds(s * rows_per_tile, rows_per_tile)])

    return k


def _deg_pass(nchunks):
    assert nchunks % 2 == 0

    @functools.partial(
        pl.kernel,
        mesh=_mesh(),
        compiler_params=pltpu.CompilerParams(use_tc_tiling_on_sc=False),
        out_type=jax.ShapeDtypeStruct((NC, ACC_ROWS, DEG_W), jnp.float32),
        scratch_types=[
            pltpu.VMEM((nchunks, CHUNK), jnp.int32),
            pltpu.VMEM((CHUNK, DEG_W), jnp.float32),
            pltpu.VMEM_SHARED((ACC_ROWS, DEG_W), jnp.float32),
            [pltpu.SemaphoreType.DMA] * 2,
        ],
    )
    def k(dst_hbm, out_hbm, dst_all, ones_v, acc, ssem):
        c = lax.axis_index("c")
        s = lax.axis_index("s")
        rows_per_tile = ACC_ROWS // NS

        def fill(val):
            def frow(i, carry):
                for jj in range(DEG_W // 16):
                    ones_v[i, pl.ds(jj * 16, 16)] = jnp.full((16,), val,
                                                             jnp.float32)
                return carry
            lax.fori_loop(0, CHUNK, frow, 0)

        fill(0.0)
        for b in range(rows_per_tile // CHUNK):
            pltpu.sync_copy(
                ones_v, acc.at[pl.ds(s * rows_per_tile + b * CHUNK, CHUNK)])
        fill(1.0)
        pltpu.sync_copy(dst_hbm.at[c, s], dst_all)
        plsc.subcore_barrier()

        def body(j, carry):
            for p in range(2):
                pltpu.async_copy(ones_v, acc.at[dst_all.at[2 * j + p]],
                                 ssem[p], add=True)
            for p in range(2):
                pltpu.make_async_copy(ones_v, acc.at[dst_all.at[0]],
                                      ssem[p]).wait()
            return carry
        lax.fori_loop(0, nchunks // 2, body, 0)

        plsc.subcore_barrier()
        pltpu.sync_copy(acc.at[pl.ds(s * rows_per_tile, rows_per_tile)],
                        out_hbm.at[c, pl.ds(s * rows_per_tile, rows_per_tile)])

    return k


def _dis_of(degs_blk):
    deg = degs_blk[0, 0] + degs_blk[0, 1] + 1.0
    return lax.rsqrt(deg)[:, None]


_DEG_SPEC = lambda: pl.BlockSpec((1, NC, ROW_BLK), lambda i: (i, 0, 0))


def _tc_stage1(degs, emb, W1):
    def body(degs_ref, emb_ref, w_ref, g_ref):
        dis = _dis_of(degs_ref)
        h = jnp.dot(emb_ref[...], w_ref[...],
                    preferred_element_type=jnp.float32)
        g_ref[0] = h[:, :NINP] * dis
        g_ref[1] = h[:, NINP:] * dis

    return pl.pallas_call(
        body,
        grid=(N // ROW_BLK,),
        in_specs=[
            _DEG_SPEC(),
            pl.BlockSpec((ROW_BLK, NINP), lambda i: (i, 0)),
            pl.BlockSpec((NINP, 2 * NINP), lambda i: (0, 0)),
        ],
        out_specs=pl.BlockSpec((NC, ROW_BLK, NINP), lambda i: (0, i, 0)),
        out_shape=jax.ShapeDtypeStruct((NC, N, NINP), jnp.float32),
    )(degs, emb, W1)


def _tc_stage2(degs, s1, g1, W2, b1):
    def body(degs_ref, s1_ref, g1_ref, w_ref, b_ref, g2_ref):
        dis = _dis_of(degs_ref)
        t0 = (s1_ref[0] + g1_ref[0]) * dis
        t1 = (s1_ref[1] + g1_ref[1]) * dis
        x1 = jnp.concatenate([t0, t1], axis=1) + b_ref[...]
        g2 = jnp.dot(x1, w_ref[...],
                     preferred_element_type=jnp.float32) * dis
        g2_ref[0] = g2
        g2_ref[1] = g2

    return pl.pallas_call(
        body,
        grid=(N // ROW_BLK,),
        in_specs=[
            _DEG_SPEC(),
            pl.BlockSpec((NC, ROW_BLK, NINP), lambda i: (0, i, 0)),
            pl.BlockSpec((NC, ROW_BLK, NINP), lambda i: (0, i, 0)),
            pl.BlockSpec((2 * NINP, NINP), lambda i: (0, 0)),
            pl.BlockSpec((1, 2 * NINP), lambda i: (0, 0)),
        ],
        out_specs=pl.BlockSpec((NC, ROW_BLK, NINP), lambda i: (0, i, 0)),
        out_shape=jax.ShapeDtypeStruct((NC, N, NINP), jnp.float32),
    )(degs, s1, g1, W2, b1)


def _tc_stage3(degs, s2, g2, b2):
    def body(degs_ref, s2_ref, g2_ref, b_ref, out_ref):
        dis = _dis_of(degs_ref)
        out_ref[...] = (s2_ref[0] + s2_ref[1] + g2_ref[0]) * dis + b_ref[...]

    return pl.pallas_call(
        body,
        grid=(N // ROW_BLK,),
        in_specs=[
            _DEG_SPEC(),
            pl.BlockSpec((NC, ROW_BLK, NINP), lambda i: (0, i, 0)),
            pl.BlockSpec((NC, ROW_BLK, NINP), lambda i: (0, i, 0)),
            pl.BlockSpec((1, NINP), lambda i: (0, 0)),
        ],
        out_specs=pl.BlockSpec((ROW_BLK, NINP), lambda i: (i, 0)),
        out_shape=jax.ShapeDtypeStruct((N, NINP), jnp.float32),
    )(degs, s2, g2, b2)


def _ceil_div(a, b):
    return (a + b - 1) // b


def _round_chunks(n):
    while n % 8 != 2:
        n += 1
    return n


@jax.jit
def kernel(edge_index, emb, W1, b1, W2, b2):
    src = edge_index[0].astype(jnp.int32)
    dst = edge_index[1].astype(jnp.int32)

    def _trash(n):
        return TRASH + (jnp.arange(n, dtype=jnp.int32) % (ACC_ROWS - N))

    e_sub = E // NS
    c1 = _round_chunks(_ceil_div(e_sub, CHUNK))
    h1 = c1 + 6
    p1 = h1 * CHUNK - e_sub
    src1 = jnp.pad(src.reshape(NS, e_sub), ((0, 0), (0, p1)))
    src1 = src1.reshape(NS, h1, CHUNK)
    dst1 = jnp.concatenate(
        [dst.reshape(NS, e_sub),
         jnp.broadcast_to(_trash(p1), (NS, p1))], axis=1)
    dst1 = dst1.reshape(NS, h1, CHUNK)
    src_l1 = jnp.stack([src1, src1 + N])
    dst_l1 = jnp.stack([dst1, dst1])

    e_tile = E // (NC * NS)
    c2 = _round_chunks(_ceil_div(e_tile, CHUNK))
    h2 = c2 + 6
    p2 = h2 * CHUNK - e_tile
    src2 = jnp.pad(src.reshape(NC, NS, e_tile),
                   ((0, 0), (0, 0), (0, p2)))
    src2 = src2.reshape(NC, NS, h2, CHUNK)
    dst2 = jnp.concatenate(
        [dst.reshape(NC, NS, e_tile),
         jnp.broadcast_to(_trash(p2), (NC, NS, p2))], axis=2)
    dst2 = dst2.reshape(NC, NS, h2, CHUNK)

    degs = _deg_pass(h2)(dst2)[:, :N, 0]
    degs = degs.reshape(NC, N // ROW_BLK, ROW_BLK).transpose(1, 0, 2)
    g1 = _tc_stage1(degs, emb, W1)
    s1 = _edge_pass(c1, NINP, CHUNK)(src_l1, dst_l1,
                                     g1.reshape(NC * N, NINP))
    g2 = _tc_stage2(degs, s1, g1, W2, b1.reshape(1, -1))
    src2d = src2 + N * jnp.arange(NC, dtype=jnp.int32)[:, None, None, None]
    s2 = _edge_pass(c2, NINP, CHUNK)(src2d, dst2,
                                     g2.reshape(NC * N, NINP))
    return _tc_stage3(degs, s2, g2, b2.reshape(1, -1))


# speedup vs baseline: 1.0540x; 1.0472x over previous
import functools

import jax
import jax.numpy as jnp
from jax import lax
from jax.experimental import pallas as pl
from jax.experimental.pallas import tpu as pltpu
from jax.experimental.pallas import tpu_sc as plsc

N = 10000
NINP = 128
E = 320000

NC, NS = 2, 16
CHUNK = 80
IBLK = 8
ACC_ROWS = 10240
TRASH = N
DEG_W = 16

ROW_BLK = 400

_mesh = lambda: plsc.VectorSubcoreMesh(core_axis_name="c", subcore_axis_name="s")


def _edge_pass(nchunks, width, chunk):
    assert nchunks % 8 == 2 and nchunks >= 10
    ngroups = (nchunks - 2) // 8
    cparams = (pltpu.CompilerParams(use_tc_tiling_on_sc=False)
               if width < 128 else None)

    @functools.partial(
        pl.kernel,
        mesh=_mesh(),
        compiler_params=cparams,
        out_type=jax.ShapeDtypeStruct((NC, ACC_ROWS, width), jnp.float32),
        scratch_types=[
            pltpu.VMEM((2, IBLK, chunk), jnp.int32),
            pltpu.VMEM((2, IBLK, chunk), jnp.int32),
            pltpu.VMEM((4, chunk, width), jnp.float32),
            pltpu.VMEM_SHARED((ACC_ROWS, width), jnp.float32),
            [pltpu.SemaphoreType.DMA] * 4,
            [pltpu.SemaphoreType.DMA] * 4,
            [pltpu.SemaphoreType.DMA] * 2,
        ],
    )
    def k(src_hbm, dst_hbm, table_hbm, out_hbm, src_blk, dst_blk, rows_v,
          acc, gsem, ssem, isem):
        c = lax.axis_index("c")
        s = lax.axis_index("s")
        rows_per_tile = ACC_ROWS // NS

        def zrow(i, carry):
            for jj in range(width // 16):
                rows_v[0, i, pl.ds(jj * 16, 16)] = jnp.zeros((16,),
                                                             jnp.float32)
            return carry
        lax.fori_loop(0, chunk, zrow, 0)
        for b in range(rows_per_tile // chunk):
            pltpu.sync_copy(rows_v.at[0],
                            acc.at[pl.ds(s * rows_per_tile + b * chunk,
                                         chunk)])
        plsc.subcore_barrier()

        def ib_start(blk, par):
            pltpu.async_copy(src_hbm.at[c, s, pl.ds(blk * IBLK, IBLK)],
                             src_blk.at[par], isem[0])
            pltpu.async_copy(dst_hbm.at[c, s, pl.ds(blk * IBLK, IBLK)],
                             dst_blk.at[par], isem[1])

        def ib_wait(par):
            pltpu.make_async_copy(src_hbm.at[c, s, pl.ds(0, IBLK)],
                                  src_blk.at[par], isem[0]).wait()
            pltpu.make_async_copy(dst_hbm.at[c, s, pl.ds(0, IBLK)],
                                  dst_blk.at[par], isem[1]).wait()

        def g_start(par, slot, b):
            pltpu.async_copy(table_hbm.at[src_blk.at[par, slot]],
                             rows_v.at[b], gsem[b])

        def g_wait(b):
            pltpu.make_async_copy(table_hbm.at[src_blk.at[0, 0]],
                                  rows_v.at[b], gsem[b]).wait()

        def s_start(par, slot, b):
            pltpu.async_copy(rows_v.at[b], acc.at[dst_blk.at[par, slot]],
                             ssem[b], add=True)

        def s_wait(b):
            pltpu.make_async_copy(rows_v.at[b], acc.at[dst_blk.at[0, 0]],
                                  ssem[b]).wait()

        zero = jnp.int32(0)
        ib_start(zero, zero)
        ib_wait(zero)
        g_start(zero, 0, 0)
        g_start(zero, 1, 1)
        g_start(zero, 2, 2); g_wait(0); s_start(zero, 0, 0)
        g_start(zero, 3, 3); g_wait(1); s_start(zero, 1, 1)

        def body(j, carry):
            parj = j & 1
            parj1 = (j + 1) & 1
            for p in range(8):
                bb = (2 + p) % 4
                if p == 0:
                    ib_start(j + 1, parj1)
                s_wait((bb + 2) % 4)
                if p == 4:
                    ib_wait(parj1)
                g_par = parj if p <= 3 else parj1
                g_start(g_par, (4 + p) % 8, (bb + 2) % 4)
                g_wait(bb)
                s_par = parj if p <= 5 else parj1
                s_start(s_par, (2 + p) % 8, bb)
            return carry
        lax.fori_loop(0, ngroups, body, 0)

        g_wait(2); g_wait(3)
        s_wait(0); s_wait(1)

        plsc.subcore_barrier()
        pltpu.sync_copy(acc.at[pl.ds(s * rows_per_tile, rows_per_tile)],
                        out_hbm.at[c, pl.ds(s * rows_per_tile, rows_per_tile)])

    return k


def _deg_pass(nchunks):
    assert nchunks % 2 == 0

    @functools.partial(
        pl.kernel,
        mesh=_mesh(),
        compiler_params=pltpu.CompilerParams(use_tc_tiling_on_sc=False),
        out_type=jax.ShapeDtypeStruct((NC, ACC_ROWS, DEG_W), jnp.float32),
        scratch_types=[
            pltpu.VMEM((nchunks, CHUNK), jnp.int32),
            pltpu.VMEM((CHUNK, DEG_W), jnp.float32),
            pltpu.VMEM_SHARED((ACC_ROWS, DEG_W), jnp.float32),
            [pltpu.SemaphoreType.DMA] * 2,
        ],
    )
    def k(dst_hbm, out_hbm, dst_all, ones_v, acc, ssem):
        c = lax.axis_index("c")
        s = lax.axis_index("s")
        rows_per_tile = ACC_ROWS // NS

        def fill(val):
            def frow(i, carry):
                for jj in range(DEG_W // 16):
                    ones_v[i, pl.ds(jj * 16, 16)] = jnp.full((16,), val,
                                                             jnp.float32)
                return carry
            lax.fori_loop(0, CHUNK, frow, 0)

        fill(0.0)
        for b in range(rows_per_tile // CHUNK):
            pltpu.sync_copy(
                ones_v, acc.at[pl.ds(s * rows_per_tile + b * CHUNK, CHUNK)])
        fill(1.0)
        pltpu.sync_copy(dst_hbm.at[c, s], dst_all)
        plsc.subcore_barrier()

        def body(j, carry):
            for p in range(2):
                pltpu.async_copy(ones_v, acc.at[dst_all.at[2 * j + p]],
                                 ssem[p], add=True)
            for p in range(2):
                pltpu.make_async_copy(ones_v, acc.at[dst_all.at[0]],
                                      ssem[p]).wait()
            return carry
        lax.fori_loop(0, nchunks // 2, body, 0)

        plsc.subcore_barrier()
        pltpu.sync_copy(acc.at[pl.ds(s * rows_per_tile, rows_per_tile)],
                        out_hbm.at[c, pl.ds(s * rows_per_tile, rows_per_tile)])

    return k


def _dis_of(degs_blk):
    deg = degs_blk[0, 0] + degs_blk[0, 1] + 1.0
    return lax.rsqrt(deg)[:, None]


_DEG_SPEC = lambda: pl.BlockSpec((1, NC, ROW_BLK), lambda i: (i, 0, 0))


def _tc_stage1(degs, emb, W1):
    def body(degs_ref, emb_ref, w_ref, g_ref):
        dis = _dis_of(degs_ref)
        h = jnp.dot(emb_ref[...], w_ref[...],
                    preferred_element_type=jnp.float32)
        g_ref[0] = h[:, :NINP] * dis
        g_ref[1] = h[:, NINP:] * dis

    return pl.pallas_call(
        body,
        grid=(N // ROW_BLK,),
        in_specs=[
            _DEG_SPEC(),
            pl.BlockSpec((ROW_BLK, NINP), lambda i: (i, 0)),
            pl.BlockSpec((NINP, 2 * NINP), lambda i: (0, 0)),
        ],
        out_specs=pl.BlockSpec((NC, ROW_BLK, NINP), lambda i: (0, i, 0)),
        out_shape=jax.ShapeDtypeStruct((NC, N, NINP), jnp.float32),
    )(degs, emb, W1)


def _tc_stage2(degs, s1, g1, W2, b1):
    def body(degs_ref, s1_ref, g1_ref, w_ref, b_ref, g2_ref):
        dis = _dis_of(degs_ref)
        t0 = (s1_ref[0] + g1_ref[0]) * dis
        t1 = (s1_ref[1] + g1_ref[1]) * dis
        x1 = jnp.concatenate([t0, t1], axis=1) + b_ref[...]
        g2 = jnp.dot(x1, w_ref[...],
                     preferred_element_type=jnp.float32) * dis
        g2_ref[0] = g2
        g2_ref[1] = g2

    return pl.pallas_call(
        body,
        grid=(N // ROW_BLK,),
        in_specs=[
            _DEG_SPEC(),
            pl.BlockSpec((NC, ROW_BLK, NINP), lambda i: (0, i, 0)),
            pl.BlockSpec((NC, ROW_BLK, NINP), lambda i: (0, i, 0)),
            pl.BlockSpec((2 * NINP, NINP), lambda i: (0, 0)),
            pl.BlockSpec((1, 2 * NINP), lambda i: (0, 0)),
        ],
        out_specs=pl.BlockSpec((NC, ROW_BLK, NINP), lambda i: (0, i, 0)),
        out_shape=jax.ShapeDtypeStruct((NC, N, NINP), jnp.float32),
    )(degs, s1, g1, W2, b1)


def _tc_stage3(degs, s2, g2, b2):
    def body(degs_ref, s2_ref, g2_ref, b_ref, out_ref):
        dis = _dis_of(degs_ref)
        out_ref[...] = (s2_ref[0] + s2_ref[1] + g2_ref[0]) * dis + b_ref[...]

    return pl.pallas_call(
        body,
        grid=(N // ROW_BLK,),
        in_specs=[
            _DEG_SPEC(),
            pl.BlockSpec((NC, ROW_BLK, NINP), lambda i: (0, i, 0)),
            pl.BlockSpec((NC, ROW_BLK, NINP), lambda i: (0, i, 0)),
            pl.BlockSpec((1, NINP), lambda i: (0, 0)),
        ],
        out_specs=pl.BlockSpec((ROW_BLK, NINP), lambda i: (i, 0)),
        out_shape=jax.ShapeDtypeStruct((N, NINP), jnp.float32),
    )(degs, s2, g2, b2)


def _ceil_div(a, b):
    return (a + b - 1) // b


def _round_chunks(n):
    while n % 8 != 2:
        n += 1
    return n


@jax.jit
def kernel(edge_index, emb, W1, b1, W2, b2):
    src = edge_index[0].astype(jnp.int32)
    dst = edge_index[1].astype(jnp.int32)

    def _trash(n):
        return TRASH + (jnp.arange(n, dtype=jnp.int32) % (ACC_ROWS - N))

    e_sub = E // NS
    c1 = _round_chunks(_ceil_div(e_sub, CHUNK))
    h1 = c1 + 6
    p1 = h1 * CHUNK - e_sub
    src1 = jnp.pad(src.reshape(NS, e_sub), ((0, 0), (0, p1)))
    src1 = src1.reshape(NS, h1, CHUNK)
    dst1 = jnp.concatenate(
        [dst.reshape(NS, e_sub),
         jnp.broadcast_to(_trash(p1), (NS, p1))], axis=1)
    dst1 = dst1.reshape(NS, h1, CHUNK)
    src_l1 = jnp.stack([src1, src1 + N])
    dst_l1 = jnp.stack([dst1, dst1])

    e_tile = E // (NC * NS)
    c2 = _round_chunks(_ceil_div(e_tile, CHUNK))
    h2 = c2 + 6
    p2 = h2 * CHUNK - e_tile
    dst2 = jnp.concatenate(
        [dst.reshape(NC, NS, e_tile),
         jnp.broadcast_to(_trash(p2), (NC, NS, p2))], axis=2)
    dst2 = dst2.reshape(NC, NS, h2, CHUNK)

    CH2 = 64
    c2b = _round_chunks(_ceil_div(e_tile, CH2))
    h2b = c2b + 6
    p2b = h2b * CH2 - e_tile
    src2 = jnp.pad(src.reshape(NC, NS, e_tile),
                   ((0, 0), (0, 0), (0, p2b)))
    src2 = src2.reshape(NC, NS, h2b, CH2)
    dst2b = jnp.concatenate(
        [dst.reshape(NC, NS, e_tile),
         jnp.broadcast_to(_trash(p2b), (NC, NS, p2b))], axis=2)
    dst2b = dst2b.reshape(NC, NS, h2b, CH2)

    degs = _deg_pass(h2)(dst2)[:, :N, 0]
    degs = degs.reshape(NC, N // ROW_BLK, ROW_BLK).transpose(1, 0, 2)
    g1 = _tc_stage1(degs, emb, W1)
    s1 = _edge_pass(c1, NINP, CHUNK)(src_l1, dst_l1,
                                     g1.reshape(NC * N, NINP))
    g2 = _tc_stage2(degs, s1, g1, W2, b1.reshape(1, -1))
    src2d = src2 + N * jnp.arange(NC, dtype=jnp.int32)[:, None, None, None]
    s2 = _edge_pass(c2b, NINP, CH2)(src2d, dst2b,
                                    g2.reshape(NC * N, NINP))
    return _tc_stage3(degs, s2, g2, b2.reshape(1, -1))


# speedup vs baseline: 1.4510x; 1.3767x over previous
import functools

import jax
import jax.numpy as jnp
from jax import lax
from jax.experimental import pallas as pl
from jax.experimental.pallas import tpu as pltpu
from jax.experimental.pallas import tpu_sc as plsc

N = 10000
NINP = 128
E = 320000

NC, NS = 2, 16
CHUNK = 80
IBLK = 8
ACC_ROWS = 10240
TRASH = N
DEG_W = 16

ROW_BLK = 400

_mesh = lambda: plsc.VectorSubcoreMesh(core_axis_name="c", subcore_axis_name="s")


def _edge_pass(nchunks, width, chunk):
    assert nchunks % 8 == 2 and nchunks >= 10
    ngroups = (nchunks - 2) // 8
    cparams = (pltpu.CompilerParams(use_tc_tiling_on_sc=False)
               if width < 128 else None)

    @functools.partial(
        pl.kernel,
        mesh=_mesh(),
        compiler_params=cparams,
        out_type=jax.ShapeDtypeStruct((NC, ACC_ROWS, width), jnp.float32),
        scratch_types=[
            pltpu.VMEM((2, IBLK, chunk), jnp.int32),
            pltpu.VMEM((2, IBLK, chunk), jnp.int32),
            pltpu.VMEM((4, chunk, width), jnp.float32),
            pltpu.VMEM_SHARED((ACC_ROWS, width), jnp.float32),
            [pltpu.SemaphoreType.DMA] * 4,
            [pltpu.SemaphoreType.DMA] * 4,
            [pltpu.SemaphoreType.DMA] * 2,
        ],
    )
    def k(src_hbm, dst_hbm, table_hbm, out_hbm, src_blk, dst_blk, rows_v,
          acc, gsem, ssem, isem):
        c = lax.axis_index("c")
        s = lax.axis_index("s")
        rows_per_tile = ACC_ROWS // NS

        def zrow(i, carry):
            for jj in range(width // 16):
                rows_v[0, i, pl.ds(jj * 16, 16)] = jnp.zeros((16,),
                                                             jnp.float32)
            return carry
        lax.fori_loop(0, chunk, zrow, 0)
        for b in range(rows_per_tile // chunk):
            pltpu.sync_copy(rows_v.at[0],
                            acc.at[pl.ds(s * rows_per_tile + b * chunk,
                                         chunk)])
        plsc.subcore_barrier()

        def ib_start(blk, par):
            pltpu.async_copy(src_hbm.at[c, s, pl.ds(blk * IBLK, IBLK)],
                             src_blk.at[par], isem[0])
            pltpu.async_copy(dst_hbm.at[c, s, pl.ds(blk * IBLK, IBLK)],
                             dst_blk.at[par], isem[1])

        def ib_wait(par):
            pltpu.make_async_copy(src_hbm.at[c, s, pl.ds(0, IBLK)],
                                  src_blk.at[par], isem[0]).wait()
            pltpu.make_async_copy(dst_hbm.at[c, s, pl.ds(0, IBLK)],
                                  dst_blk.at[par], isem[1]).wait()

        def g_start(par, slot, b):
            pltpu.async_copy(table_hbm.at[src_blk.at[par, slot]],
                             rows_v.at[b], gsem[b])

        def g_wait(b):
            pltpu.make_async_copy(table_hbm.at[src_blk.at[0, 0]],
                                  rows_v.at[b], gsem[b]).wait()

        def s_start(par, slot, b):
            pltpu.async_copy(rows_v.at[b], acc.at[dst_blk.at[par, slot]],
                             ssem[b], add=True)

        def s_wait(b):
            pltpu.make_async_copy(rows_v.at[b], acc.at[dst_blk.at[0, 0]],
                                  ssem[b]).wait()

        zero = jnp.int32(0)
        ib_start(zero, zero)
        ib_wait(zero)
        g_start(zero, 0, 0)
        g_start(zero, 1, 1)
        g_start(zero, 2, 2); g_wait(0); s_start(zero, 0, 0)
        g_start(zero, 3, 3); g_wait(1); s_start(zero, 1, 1)

        def body(j, carry):
            parj = j & 1
            parj1 = (j + 1) & 1
            for p in range(8):
                bb = (2 + p) % 4
                if p == 0:
                    ib_start(j + 1, parj1)
                s_wait((bb + 2) % 4)
                if p == 4:
                    ib_wait(parj1)
                g_par = parj if p <= 3 else parj1
                g_start(g_par, (4 + p) % 8, (bb + 2) % 4)
                g_wait(bb)
                s_par = parj if p <= 5 else parj1
                s_start(s_par, (2 + p) % 8, bb)
            return carry
        lax.fori_loop(0, ngroups, body, 0)

        g_wait(2); g_wait(3)
        s_wait(0); s_wait(1)

        plsc.subcore_barrier()
        pltpu.sync_copy(acc.at[pl.ds(s * rows_per_tile, rows_per_tile)],
                        out_hbm.at[c, pl.ds(s * rows_per_tile, rows_per_tile)])

    return k


def _deg_pass(nchunks):
    assert nchunks % 2 == 0

    @functools.partial(
        pl.kernel,
        mesh=_mesh(),
        compiler_params=pltpu.CompilerParams(use_tc_tiling_on_sc=False),
        out_type=jax.ShapeDtypeStruct((NC, ACC_ROWS, DEG_W), jnp.float32),
        scratch_types=[
            pltpu.VMEM((nchunks, CHUNK), jnp.int32),
            pltpu.VMEM((CHUNK, DEG_W), jnp.float32),
            pltpu.VMEM_SHARED((ACC_ROWS, DEG_W), jnp.float32),
            [pltpu.SemaphoreType.DMA] * 2,
        ],
    )
    def k(dst_hbm, out_hbm, dst_all, ones_v, acc, ssem):
        c = lax.axis_index("c")
        s = lax.axis_index("s")
        rows_per_tile = ACC_ROWS // NS

        def fill(val):
            def frow(i, carry):
                for jj in range(DEG_W // 16):
                    ones_v[i, pl.ds(jj * 16, 16)] = jnp.full((16,), val,
                                                             jnp.float32)
                return carry
            lax.fori_loop(0, CHUNK, frow, 0)

        fill(0.0)
        for b in range(rows_per_tile // CHUNK):
            pltpu.sync_copy(
                ones_v, acc.at[pl.ds(s * rows_per_tile + b * CHUNK, CHUNK)])
        fill(1.0)
        pltpu.sync_copy(dst_hbm.at[c, s], dst_all)
        plsc.subcore_barrier()

        def body(j, carry):
            for p in range(2):
                pltpu.async_copy(ones_v, acc.at[dst_all.at[2 * j + p]],
                                 ssem[p], add=True)
            for p in range(2):
                pltpu.make_async_copy(ones_v, acc.at[dst_all.at[0]],
                                      ssem[p]).wait()
            return carry
        lax.fori_loop(0, nchunks // 2, body, 0)

        plsc.subcore_barrier()
        pltpu.sync_copy(acc.at[pl.ds(s * rows_per_tile, rows_per_tile)],
                        out_hbm.at[c, pl.ds(s * rows_per_tile, rows_per_tile)])

    return k


def _dis_of(degs_blk):
    deg = degs_blk[0, 0] + degs_blk[0, 1] + 1.0
    return lax.rsqrt(deg)[:, None]


_DEG_SPEC = lambda: pl.BlockSpec((1, NC, ROW_BLK), lambda i: (i, 0, 0))


def _tc_stage1(degs, emb, W1):
    def body(degs_ref, emb_ref, w_ref, g_ref):
        dis = _dis_of(degs_ref)
        h = jnp.dot(emb_ref[...], w_ref[...],
                    preferred_element_type=jnp.float32)
        g_ref[0] = h[:, :NINP] * dis
        g_ref[1] = h[:, NINP:] * dis

    return pl.pallas_call(
        body,
        grid=(N // ROW_BLK,),
        in_specs=[
            _DEG_SPEC(),
            pl.BlockSpec((ROW_BLK, NINP), lambda i: (i, 0)),
            pl.BlockSpec((NINP, 2 * NINP), lambda i: (0, 0)),
        ],
        out_specs=pl.BlockSpec((NC, ROW_BLK, NINP), lambda i: (0, i, 0)),
        out_shape=jax.ShapeDtypeStruct((NC, N, NINP), jnp.float32),
    )(degs, emb, W1)


def _tc_stage2(degs, s1, g1, W2, b1):
    def body(degs_ref, s1_ref, g1_ref, w_ref, b_ref, g2_ref):
        dis = _dis_of(degs_ref)
        t0 = (s1_ref[0] + g1_ref[0]) * dis
        t1 = (s1_ref[1] + g1_ref[1]) * dis
        x1 = jnp.concatenate([t0, t1], axis=1) + b_ref[...]
        g2 = jnp.dot(x1, w_ref[...],
                     preferred_element_type=jnp.float32) * dis
        g2_ref[0] = g2
        g2_ref[1] = g2

    return pl.pallas_call(
        body,
        grid=(N // ROW_BLK,),
        in_specs=[
            _DEG_SPEC(),
            pl.BlockSpec((NC, ROW_BLK, NINP), lambda i: (0, i, 0)),
            pl.BlockSpec((NC, ROW_BLK, NINP), lambda i: (0, i, 0)),
            pl.BlockSpec((2 * NINP, NINP), lambda i: (0, 0)),
            pl.BlockSpec((1, 2 * NINP), lambda i: (0, 0)),
        ],
        out_specs=pl.BlockSpec((NC, ROW_BLK, NINP), lambda i: (0, i, 0)),
        out_shape=jax.ShapeDtypeStruct((NC, N, NINP), jnp.float32),
    )(degs, s1, g1, W2, b1)


def _tc_stage3(degs, s2, g2, b2):
    def body(degs_ref, s2_ref, g2_ref, b_ref, out_ref):
        dis = _dis_of(degs_ref)
        out_ref[...] = (s2_ref[0] + s2_ref[1] + g2_ref[0]) * dis + b_ref[...]

    return pl.pallas_call(
        body,
        grid=(N // ROW_BLK,),
        in_specs=[
            _DEG_SPEC(),
            pl.BlockSpec((NC, ROW_BLK, NINP), lambda i: (0, i, 0)),
            pl.BlockSpec((NC, ROW_BLK, NINP), lambda i: (0, i, 0)),
            pl.BlockSpec((1, NINP), lambda i: (0, 0)),
        ],
        out_specs=pl.BlockSpec((ROW_BLK, NINP), lambda i: (i, 0)),
        out_shape=jax.ShapeDtypeStruct((N, NINP), jnp.float32),
    )(degs, s2, g2, b2)


def _ceil_div(a, b):
    return (a + b - 1) // b


def _round_chunks(n):
    while n % 8 != 2:
        n += 1
    return n


@jax.jit
def kernel(edge_index, emb, W1, b1, W2, b2):
    src = edge_index[0].astype(jnp.int32)
    dst = edge_index[1].astype(jnp.int32)

    def _trash(n):
        return TRASH + (jnp.arange(n, dtype=jnp.int32) % (ACC_ROWS - N))

    e_sub = E // NS
    c1 = _round_chunks(_ceil_div(e_sub, CHUNK))
    h1 = c1 + 6
    p1 = h1 * CHUNK - e_sub
    src1 = jnp.pad(src.reshape(NS, e_sub), ((0, 0), (0, p1)))
    src1 = src1.reshape(NS, h1, CHUNK)
    dst1 = jnp.concatenate(
        [dst.reshape(NS, e_sub),
         jnp.broadcast_to(_trash(p1), (NS, p1))], axis=1)
    dst1 = dst1.reshape(NS, h1, CHUNK)
    src_l1 = jnp.stack([src1, src1 + N])
    dst_l1 = jnp.stack([dst1, dst1])

    e_tile = E // (NC * NS)
    c2 = _round_chunks(_ceil_div(e_tile, CHUNK))
    h2 = c2 + 6
    p2 = h2 * CHUNK - e_tile
    dst2 = jnp.concatenate(
        [dst.reshape(NC, NS, e_tile),
         jnp.broadcast_to(_trash(p2), (NC, NS, p2))], axis=2)
    dst2 = dst2.reshape(NC, NS, h2, CHUNK)

    CH2 = 32
    c2b = _round_chunks(_ceil_div(e_tile, CH2))
    h2b = c2b + 6
    p2b = h2b * CH2 - e_tile
    src2 = jnp.pad(src.reshape(NC, NS, e_tile),
                   ((0, 0), (0, 0), (0, p2b)))
    src2 = src2.reshape(NC, NS, h2b, CH2)
    dst2b = jnp.concatenate(
        [dst.reshape(NC, NS, e_tile),
         jnp.broadcast_to(_trash(p2b), (NC, NS, p2b))], axis=2)
    dst2b = dst2b.reshape(NC, NS, h2b, CH2)

    degs = _deg_pass(h2)(dst2)[:, :N, 0]
    degs = degs.reshape(NC, N // ROW_BLK, ROW_BLK).transpose(1, 0, 2)
    g1 = _tc_stage1(degs, emb, W1)
    s1 = _edge_pass(c1, NINP, CHUNK)(src_l1, dst_l1,
                                     g1.reshape(NC * N, NINP))
    g2 = _tc_stage2(degs, s1, g1, W2, b1.reshape(1, -1))
    src2d = src2 + N * jnp.arange(NC, dtype=jnp.int32)[:, None, None, None]
    s2 = _edge_pass(c2b, NINP, CH2)(src2d, dst2b,
                                    g2.reshape(NC * N, NINP))
    return _tc_stage3(degs, s2, g2, b2.reshape(1, -1))


# speedup vs baseline: 1.4740x; 1.0158x over previous
import functools

import jax
import jax.numpy as jnp
from jax import lax
from jax.experimental import pallas as pl
from jax.experimental.pallas import tpu as pltpu
from jax.experimental.pallas import tpu_sc as plsc

N = 10000
NINP = 128
E = 320000

NC, NS = 2, 16
CHUNK = 80
IBLK = 8
ACC_ROWS = 10240
TRASH = N
DEG_W = 16

ROW_BLK = 400

_mesh = lambda: plsc.VectorSubcoreMesh(core_axis_name="c", subcore_axis_name="s")


def _edge_pass(nchunks, width, chunk):
    assert nchunks % 8 == 2 and nchunks >= 10
    ngroups = (nchunks - 2) // 8
    cparams = (pltpu.CompilerParams(use_tc_tiling_on_sc=False)
               if width < 128 else None)

    @functools.partial(
        pl.kernel,
        mesh=_mesh(),
        compiler_params=cparams,
        out_type=jax.ShapeDtypeStruct((NC, ACC_ROWS, width), jnp.float32),
        scratch_types=[
            pltpu.VMEM((2, IBLK, chunk), jnp.int32),
            pltpu.VMEM((2, IBLK, chunk), jnp.int32),
            pltpu.VMEM((4, chunk, width), jnp.float32),
            pltpu.VMEM_SHARED((ACC_ROWS, width), jnp.float32),
            [pltpu.SemaphoreType.DMA] * 4,
            [pltpu.SemaphoreType.DMA] * 4,
            [pltpu.SemaphoreType.DMA] * 2,
        ],
    )
    def k(src_hbm, dst_hbm, table_hbm, out_hbm, src_blk, dst_blk, rows_v,
          acc, gsem, ssem, isem):
        c = lax.axis_index("c")
        s = lax.axis_index("s")
        rows_per_tile = ACC_ROWS // NS

        def zrow(i, carry):
            for jj in range(width // 16):
                rows_v[0, i, pl.ds(jj * 16, 16)] = jnp.zeros((16,),
                                                             jnp.float32)
            return carry
        lax.fori_loop(0, chunk, zrow, 0)
        for b in range(rows_per_tile // chunk):
            pltpu.sync_copy(rows_v.at[0],
                            acc.at[pl.ds(s * rows_per_tile + b * chunk,
                                         chunk)])
        plsc.subcore_barrier()

        def ib_start(blk, par):
            pltpu.async_copy(src_hbm.at[c, s, pl.ds(blk * IBLK, IBLK)],
                             src_blk.at[par], isem[0])
            pltpu.async_copy(dst_hbm.at[c, s, pl.ds(blk * IBLK, IBLK)],
                             dst_blk.at[par], isem[1])

        def ib_wait(par):
            pltpu.make_async_copy(src_hbm.at[c, s, pl.ds(0, IBLK)],
                                  src_blk.at[par], isem[0]).wait()
            pltpu.make_async_copy(dst_hbm.at[c, s, pl.ds(0, IBLK)],
                                  dst_blk.at[par], isem[1]).wait()

        def g_start(par, slot, b):
            pltpu.async_copy(table_hbm.at[src_blk.at[par, slot]],
                             rows_v.at[b], gsem[b])

        def g_wait(b):
            pltpu.make_async_copy(table_hbm.at[src_blk.at[0, 0]],
                                  rows_v.at[b], gsem[b]).wait()

        def s_start(par, slot, b):
            pltpu.async_copy(rows_v.at[b], acc.at[dst_blk.at[par, slot]],
                             ssem[b], add=True)

        def s_wait(b):
            pltpu.make_async_copy(rows_v.at[b], acc.at[dst_blk.at[0, 0]],
                                  ssem[b]).wait()

        zero = jnp.int32(0)
        ib_start(zero, zero)
        ib_wait(zero)
        g_start(zero, 0, 0)
        g_start(zero, 1, 1)
        g_start(zero, 2, 2); g_wait(0); s_start(zero, 0, 0)
        g_start(zero, 3, 3); g_wait(1); s_start(zero, 1, 1)

        def body(j, carry):
            parj = j & 1
            parj1 = (j + 1) & 1
            for p in range(8):
                bb = (2 + p) % 4
                if p == 0:
                    ib_start(j + 1, parj1)
                s_wait((bb + 2) % 4)
                if p == 4:
                    ib_wait(parj1)
                g_par = parj if p <= 3 else parj1
                g_start(g_par, (4 + p) % 8, (bb + 2) % 4)
                g_wait(bb)
                s_par = parj if p <= 5 else parj1
                s_start(s_par, (2 + p) % 8, bb)
            return carry
        lax.fori_loop(0, ngroups, body, 0)

        g_wait(2); g_wait(3)
        s_wait(0); s_wait(1)

        plsc.subcore_barrier()
        pltpu.sync_copy(acc.at[pl.ds(s * rows_per_tile, rows_per_tile)],
                        out_hbm.at[c, pl.ds(s * rows_per_tile, rows_per_tile)])

    return k


def _deg_pass(nchunks):
    assert nchunks % 2 == 0

    @functools.partial(
        pl.kernel,
        mesh=_mesh(),
        compiler_params=pltpu.CompilerParams(use_tc_tiling_on_sc=False),
        out_type=jax.ShapeDtypeStruct((NC, ACC_ROWS, DEG_W), jnp.float32),
        scratch_types=[
            pltpu.VMEM((nchunks, CHUNK), jnp.int32),
            pltpu.VMEM((CHUNK, DEG_W), jnp.float32),
            pltpu.VMEM_SHARED((ACC_ROWS, DEG_W), jnp.float32),
            [pltpu.SemaphoreType.DMA] * 2,
        ],
    )
    def k(dst_hbm, out_hbm, dst_all, ones_v, acc, ssem):
        c = lax.axis_index("c")
        s = lax.axis_index("s")
        rows_per_tile = ACC_ROWS // NS

        def fill(val):
            def frow(i, carry):
                for jj in range(DEG_W // 16):
                    ones_v[i, pl.ds(jj * 16, 16)] = jnp.full((16,), val,
                                                             jnp.float32)
                return carry
            lax.fori_loop(0, CHUNK, frow, 0)

        fill(0.0)
        for b in range(rows_per_tile // CHUNK):
            pltpu.sync_copy(
                ones_v, acc.at[pl.ds(s * rows_per_tile + b * CHUNK, CHUNK)])
        fill(1.0)
        pltpu.sync_copy(dst_hbm.at[c, s], dst_all)
        plsc.subcore_barrier()

        def body(j, carry):
            for p in range(2):
                pltpu.async_copy(ones_v, acc.at[dst_all.at[2 * j + p]],
                                 ssem[p], add=True)
            for p in range(2):
                pltpu.make_async_copy(ones_v, acc.at[dst_all.at[0]],
                                      ssem[p]).wait()
            return carry
        lax.fori_loop(0, nchunks // 2, body, 0)

        plsc.subcore_barrier()
        pltpu.sync_copy(acc.at[pl.ds(s * rows_per_tile, rows_per_tile)],
                        out_hbm.at[c, pl.ds(s * rows_per_tile, rows_per_tile)])

    return k


def _dis_of(degs_blk):
    deg = degs_blk[0, 0] + degs_blk[0, 1] + 1.0
    return lax.rsqrt(deg)[:, None]


_DEG_SPEC = lambda: pl.BlockSpec((1, NC, ROW_BLK), lambda i: (i, 0, 0))


def _tc_stage1(degs, emb, W1):
    def body(degs_ref, emb_ref, w_ref, g_ref):
        dis = _dis_of(degs_ref)
        h = jnp.dot(emb_ref[...], w_ref[...],
                    preferred_element_type=jnp.float32)
        g_ref[0] = h[:, :NINP] * dis
        g_ref[1] = h[:, NINP:] * dis

    return pl.pallas_call(
        body,
        grid=(N // ROW_BLK,),
        in_specs=[
            _DEG_SPEC(),
            pl.BlockSpec((ROW_BLK, NINP), lambda i: (i, 0)),
            pl.BlockSpec((NINP, 2 * NINP), lambda i: (0, 0)),
        ],
        out_specs=pl.BlockSpec((NC, ROW_BLK, NINP), lambda i: (0, i, 0)),
        out_shape=jax.ShapeDtypeStruct((NC, N, NINP), jnp.float32),
    )(degs, emb, W1)


def _tc_stage2(degs, s1, g1, W2, b1):
    def body(degs_ref, s1_ref, g1_ref, w_ref, b_ref, g2_ref):
        dis = _dis_of(degs_ref)
        t0 = (s1_ref[0] + g1_ref[0]) * dis
        t1 = (s1_ref[1] + g1_ref[1]) * dis
        x1 = jnp.concatenate([t0, t1], axis=1) + b_ref[...]
        g2 = jnp.dot(x1, w_ref[...],
                     preferred_element_type=jnp.float32) * dis
        g2_ref[0] = g2
        g2_ref[1] = g2

    return pl.pallas_call(
        body,
        grid=(N // ROW_BLK,),
        in_specs=[
            _DEG_SPEC(),
            pl.BlockSpec((NC, ROW_BLK, NINP), lambda i: (0, i, 0)),
            pl.BlockSpec((NC, ROW_BLK, NINP), lambda i: (0, i, 0)),
            pl.BlockSpec((2 * NINP, NINP), lambda i: (0, 0)),
            pl.BlockSpec((1, 2 * NINP), lambda i: (0, 0)),
        ],
        out_specs=pl.BlockSpec((NC, ROW_BLK, NINP), lambda i: (0, i, 0)),
        out_shape=jax.ShapeDtypeStruct((NC, N, NINP), jnp.float32),
    )(degs, s1, g1, W2, b1)


def _tc_stage3(degs, s2, g2, b2):
    def body(degs_ref, s2_ref, g2_ref, b_ref, out_ref):
        dis = _dis_of(degs_ref)
        out_ref[...] = (s2_ref[0] + s2_ref[1] + g2_ref[0]) * dis + b_ref[...]

    return pl.pallas_call(
        body,
        grid=(N // ROW_BLK,),
        in_specs=[
            _DEG_SPEC(),
            pl.BlockSpec((NC, ROW_BLK, NINP), lambda i: (0, i, 0)),
            pl.BlockSpec((NC, ROW_BLK, NINP), lambda i: (0, i, 0)),
            pl.BlockSpec((1, NINP), lambda i: (0, 0)),
        ],
        out_specs=pl.BlockSpec((ROW_BLK, NINP), lambda i: (i, 0)),
        out_shape=jax.ShapeDtypeStruct((N, NINP), jnp.float32),
    )(degs, s2, g2, b2)


def _ceil_div(a, b):
    return (a + b - 1) // b


def _round_chunks(n):
    while n % 8 != 2:
        n += 1
    return n


@jax.jit
def kernel(edge_index, emb, W1, b1, W2, b2):
    src = edge_index[0].astype(jnp.int32)
    dst = edge_index[1].astype(jnp.int32)

    def _trash(n):
        return TRASH + (jnp.arange(n, dtype=jnp.int32) % (ACC_ROWS - N))

    e_sub = E // NS
    CH1 = 32
    c1 = _round_chunks(_ceil_div(e_sub, CH1))
    h1 = c1 + 6
    p1 = h1 * CH1 - e_sub
    src1 = jnp.pad(src.reshape(NS, e_sub), ((0, 0), (0, p1)))
    src1 = src1.reshape(NS, h1, CH1)
    dst1 = jnp.concatenate(
        [dst.reshape(NS, e_sub),
         jnp.broadcast_to(_trash(p1), (NS, p1))], axis=1)
    dst1 = dst1.reshape(NS, h1, CH1)
    src_l1 = jnp.stack([src1, src1 + N])
    dst_l1 = jnp.stack([dst1, dst1])

    e_tile = E // (NC * NS)
    c2 = _round_chunks(_ceil_div(e_tile, CHUNK))
    h2 = c2 + 6
    p2 = h2 * CHUNK - e_tile
    dst2 = jnp.concatenate(
        [dst.reshape(NC, NS, e_tile),
         jnp.broadcast_to(_trash(p2), (NC, NS, p2))], axis=2)
    dst2 = dst2.reshape(NC, NS, h2, CHUNK)

    CH2 = 32
    c2b = _round_chunks(_ceil_div(e_tile, CH2))
    h2b = c2b + 6
    p2b = h2b * CH2 - e_tile
    src2 = jnp.pad(src.reshape(NC, NS, e_tile),
                   ((0, 0), (0, 0), (0, p2b)))
    src2 = src2.reshape(NC, NS, h2b, CH2)
    dst2b = jnp.concatenate(
        [dst.reshape(NC, NS, e_tile),
         jnp.broadcast_to(_trash(p2b), (NC, NS, p2b))], axis=2)
    dst2b = dst2b.reshape(NC, NS, h2b, CH2)

    degs = _deg_pass(h2)(dst2)[:, :N, 0]
    degs = degs.reshape(NC, N // ROW_BLK, ROW_BLK).transpose(1, 0, 2)
    g1 = _tc_stage1(degs, emb, W1)
    s1 = _edge_pass(c1, NINP, CH1)(src_l1, dst_l1,
                                   g1.reshape(NC * N, NINP))
    g2 = _tc_stage2(degs, s1, g1, W2, b1.reshape(1, -1))
    src2d = src2 + N * jnp.arange(NC, dtype=jnp.int32)[:, None, None, None]
    s2 = _edge_pass(c2b, NINP, CH2)(src2d, dst2b,
                                    g2.reshape(NC * N, NINP))
    return _tc_stage3(degs, s2, g2, b2.reshape(1, -1))
